# baseline (speedup 1.0000x reference)
import functools

import jax
import jax.numpy as jnp
from jax import lax
from jax.experimental import pallas as pl
from jax.experimental.pallas import tpu as pltpu
from jax.experimental.pallas import tpu_sc as plsc

B = 16384
ND = 16
NC = 2
NS = 16
NW = NC * NS
BPW = B // NW
WAVE = 8
NWAVE = BPW // WAVE
TILE = 128


def _softplus_neg(d):
    q = jnp.exp(-jnp.abs(d))
    t = q / (2.0 + q)
    t2 = t * t
    poly = 1.0 + t2 * (
        (1.0 / 3.0) + t2 * ((1.0 / 5.0) + t2 * ((1.0 / 7.0) + t2 * (1.0 / 9.0)))
    )
    return jnp.maximum(-d, 0.0) + 2.0 * t * poly


def _sc_body(user_t, item_t, uids_hbm, pids_hbm, nids_hbm, part_hbm,
             idx_u, idx_p, idx_n,
             u0, p0, n0, u1, p1, n1, s_v, sem0, sem1):
    wid = lax.axis_index("s") * NC + lax.axis_index("c")
    base = wid * BPW

    pltpu.sync_copy(uids_hbm.at[pl.ds(base, BPW)], idx_u)
    pltpu.sync_copy(pids_hbm.at[pl.ds(base, BPW)], idx_p)
    pltpu.sync_copy(nids_hbm.at[pl.ds(base, BPW)], idx_n)

    lane = lax.iota(jnp.int32, ND)
    bufs = ((u0, p0, n0, sem0), (u1, p1, n1, sem1))
    NPAIR = NWAVE // 2

    def load_blk(pi):
        sl = pl.ds(pi * ND, ND)
        return (idx_u[sl] & ~(TILE - 1),
                idx_p[sl] & ~(TILE - 1),
                idx_n[sl] & ~(TILE - 1))

    def fire(blks, half, bs):
        blk_u, blk_p, blk_n = blks
        ub, pb, nb, sem = bs
        for k in range(WAVE):
            j = half * WAVE + k
            dsl = pl.ds(k * ND, ND)
            pltpu.async_copy(
                user_t.at[:, pl.ds(pl.multiple_of(blk_u[j], TILE), TILE)],
                ub.at[dsl, :], sem)
            pltpu.async_copy(
                item_t.at[:, pl.ds(pl.multiple_of(blk_p[j], TILE), TILE)],
                pb.at[dsl, :], sem)
            pltpu.async_copy(
                item_t.at[:, pl.ds(pl.multiple_of(blk_n[j], TILE), TILE)],
                nb.at[dsl, :], sem)

    def drain(bs):
        ub, pb, nb, sem = bs
        for k in range(WAVE):
            dsl = pl.ds(k * ND, ND)
            pltpu.make_async_copy(user_t.at[:, pl.ds(0, TILE)],
                                  ub.at[dsl, :], sem).wait()
            pltpu.make_async_copy(item_t.at[:, pl.ds(0, TILE)],
                                  pb.at[dsl, :], sem).wait()
            pltpu.make_async_copy(item_t.at[:, pl.ds(0, TILE)],
                                  nb.at[dsl, :], sem).wait()

    def half_scores(subs, half, bs, d):
        sub_u, sub_p, sub_n = subs
        ub, pb, nb, _ = bs
        for k in range(WAVE):
            j = half * WAVE + k
            rows = k * ND + lane
            u = plsc.load_gather(ub, [rows, jnp.full((ND,), sub_u[j])])
            p = plsc.load_gather(pb, [rows, jnp.full((ND,), sub_p[j])])
            n = plsc.load_gather(nb, [rows, jnp.full((ND,), sub_n[j])])
            d = jnp.where(lane == j, jnp.sum(u * (p - n)), d)
        return d

    fire(load_blk(0), 0, bufs[0])
    fire(load_blk(0), 1, bufs[1])

    def pair(i, s_acc):
        sl = pl.ds(i * ND, ND)
        subs = (idx_u[sl] & (TILE - 1),
                idx_p[sl] & (TILE - 1),
                idx_n[sl] & (TILE - 1))
        nxt = load_blk((i + 1) & (NPAIR - 1))
        drain(bufs[0])
        d = half_scores(subs, 0, bufs[0], jnp.zeros((ND,), jnp.float32))
        fire(nxt, 0, bufs[0])
        drain(bufs[1])
        d = half_scores(subs, 1, bufs[1], d)
        fire(nxt, 1, bufs[1])
        return s_acc + _softplus_neg(d)

    s_acc = lax.fori_loop(0, NPAIR, pair, jnp.zeros((ND,), jnp.float32),
                          unroll=False)
    drain(bufs[0])
    drain(bufs[1])
    s_v[...] = s_acc
    pltpu.sync_copy(s_v, part_hbm.at[wid])


@jax.jit
def _sc_partials(user_emb, item_emb, uids, pids, nids):
    user_t = user_emb.T
    item_t = item_emb.T
    mesh = plsc.VectorSubcoreMesh(core_axis_name="c", subcore_axis_name="s")
    kfn = pl.kernel(
        _sc_body,
        out_type=jax.ShapeDtypeStruct((NW, ND), jnp.float32),
        mesh=mesh,
        scratch_types=[
            pltpu.VMEM((BPW,), jnp.int32),
            pltpu.VMEM((BPW,), jnp.int32),
            pltpu.VMEM((BPW,), jnp.int32),
            pltpu.VMEM((WAVE * ND, TILE), jnp.float32),
            pltpu.VMEM((WAVE * ND, TILE), jnp.float32),
            pltpu.VMEM((WAVE * ND, TILE), jnp.float32),
            pltpu.VMEM((WAVE * ND, TILE), jnp.float32),
            pltpu.VMEM((WAVE * ND, TILE), jnp.float32),
            pltpu.VMEM((WAVE * ND, TILE), jnp.float32),
            pltpu.VMEM((ND,), jnp.float32),
            pltpu.SemaphoreType.DMA,
            pltpu.SemaphoreType.DMA,
        ],
        compiler_params=pltpu.CompilerParams(needs_layout_passes=False),
    )
    return kfn(user_t, item_t, uids, pids, nids)


def _loss_body(part_ref, out_ref):
    out_ref[0, 0] = jnp.sum(part_ref[...]) * (1.0 / B)


@jax.jit
def _tc_loss(part):
    return pl.pallas_call(
        _loss_body,
        out_shape=jax.ShapeDtypeStruct((1, 1), jnp.float32),
        out_specs=pl.BlockSpec(memory_space=pltpu.SMEM),
    )(part)


def kernel(X, user_emb, item_emb):
    uids = X[:, 0]
    pids = X[:, 1]
    nids = X[:, 2]
    part = _sc_partials(user_emb, item_emb, uids, pids, nids)
    loss = _tc_loss(part)
    return loss.reshape(())


# speedup vs baseline: 1.0258x; 1.0258x over previous
import functools

import jax
import jax.numpy as jnp
from jax import lax
from jax.experimental import pallas as pl
from jax.experimental.pallas import tpu as pltpu
from jax.experimental.pallas import tpu_sc as plsc

B = 16384
ND = 16
NC = 2
NS = 16
NW = NC * NS
BPW = B // NW
WAVE = 16
NWAVE = BPW // WAVE
TILE = 128


def _softplus_neg(d):
    q = jnp.exp(-jnp.abs(d))
    t = q / (2.0 + q)
    t2 = t * t
    poly = 1.0 + t2 * (
        (1.0 / 3.0) + t2 * ((1.0 / 5.0) + t2 * ((1.0 / 7.0) + t2 * (1.0 / 9.0)))
    )
    return jnp.maximum(-d, 0.0) + 2.0 * t * poly


def _sc_body(user_t, item_t, uids_hbm, pids_hbm, nids_hbm, part_hbm,
             idx_u, idx_p, idx_n, u_blk, p_blk, n_blk, s_v, sem):
    wid = lax.axis_index("s") * NC + lax.axis_index("c")
    base = wid * BPW

    pltpu.sync_copy(uids_hbm.at[pl.ds(base, BPW)], idx_u)
    pltpu.sync_copy(pids_hbm.at[pl.ds(base, BPW)], idx_p)
    pltpu.sync_copy(nids_hbm.at[pl.ds(base, BPW)], idx_n)

    lane = lax.iota(jnp.int32, ND)

    def wave(w, s_acc):
        sl16 = pl.ds(w * WAVE, WAVE)
        ids_u = idx_u[sl16]
        ids_p = idx_p[sl16]
        ids_n = idx_n[sl16]
        blk_u = ids_u & ~(TILE - 1)
        blk_p = ids_p & ~(TILE - 1)
        blk_n = ids_n & ~(TILE - 1)
        sub_u = ids_u & (TILE - 1)
        sub_p = ids_p & (TILE - 1)
        sub_n = ids_n & (TILE - 1)

        for k in range(WAVE):
            dsl = pl.ds(k * ND, ND)
            pltpu.async_copy(
                user_t.at[:, pl.ds(pl.multiple_of(blk_u[k], TILE), TILE)],
                u_blk.at[dsl, :], sem)
            pltpu.async_copy(
                item_t.at[:, pl.ds(pl.multiple_of(blk_p[k], TILE), TILE)],
                p_blk.at[dsl, :], sem)
            pltpu.async_copy(
                item_t.at[:, pl.ds(pl.multiple_of(blk_n[k], TILE), TILE)],
                n_blk.at[dsl, :], sem)
        for k in range(WAVE):
            dsl = pl.ds(k * ND, ND)
            pltpu.make_async_copy(user_t.at[:, pl.ds(0, TILE)],
                                  u_blk.at[dsl, :], sem).wait()
            pltpu.make_async_copy(item_t.at[:, pl.ds(0, TILE)],
                                  p_blk.at[dsl, :], sem).wait()
            pltpu.make_async_copy(item_t.at[:, pl.ds(0, TILE)],
                                  n_blk.at[dsl, :], sem).wait()

        d = jnp.zeros((ND,), jnp.float32)
        for k in range(WAVE):
            rows = k * ND + lane
            u = plsc.load_gather(u_blk, [rows, jnp.full((ND,), sub_u[k])])
            p = plsc.load_gather(p_blk, [rows, jnp.full((ND,), sub_p[k])])
            n = plsc.load_gather(n_blk, [rows, jnp.full((ND,), sub_n[k])])
            d = jnp.where(lane == k, jnp.sum(u * (p - n)), d)
        return s_acc + _softplus_neg(d)

    s_acc = lax.fori_loop(0, NWAVE, wave, jnp.zeros((ND,), jnp.float32),
                          unroll=False)
    s_v[...] = s_acc
    pltpu.sync_copy(s_v, part_hbm.at[wid])


@jax.jit
def _sc_partials(user_emb, item_emb, uids, pids, nids):
    user_t = user_emb.T
    item_t = item_emb.T
    mesh = plsc.VectorSubcoreMesh(core_axis_name="c", subcore_axis_name="s")
    kfn = pl.kernel(
        _sc_body,
        out_type=jax.ShapeDtypeStruct((NW, ND), jnp.float32),
        mesh=mesh,
        scratch_types=[
            pltpu.VMEM((BPW,), jnp.int32),
            pltpu.VMEM((BPW,), jnp.int32),
            pltpu.VMEM((BPW,), jnp.int32),
            pltpu.VMEM((WAVE * ND, TILE), jnp.float32),
            pltpu.VMEM((WAVE * ND, TILE), jnp.float32),
            pltpu.VMEM((WAVE * ND, TILE), jnp.float32),
            pltpu.VMEM((ND,), jnp.float32),
            pltpu.SemaphoreType.DMA,
        ],
        compiler_params=pltpu.CompilerParams(needs_layout_passes=False),
    )
    return kfn(user_t, item_t, uids, pids, nids)


def _loss_body(part_ref, out_ref):
    out_ref[0, 0] = jnp.sum(part_ref[...]) * (1.0 / B)


@jax.jit
def _tc_loss(part):
    return pl.pallas_call(
        _loss_body,
        out_shape=jax.ShapeDtypeStruct((1, 1), jnp.float32),
        out_specs=pl.BlockSpec(memory_space=pltpu.SMEM),
    )(part)


def kernel(X, user_emb, item_emb):
    uids = X[:, 0]
    pids = X[:, 1]
    nids = X[:, 2]
    part = _sc_partials(user_emb, item_emb, uids, pids, nids)
    loss = _tc_loss(part)
    return loss.reshape(())
